# baseline (speedup 1.0000x reference)
import numpy as np
import jax
import jax.numpy as jnp
from jax.experimental import pallas as pl
from jax.experimental.pallas import tpu as pltpu

_HASH_MOD = 2**31 - 1


def _pymod(s):
    m = jax.lax.rem(s, jnp.int32(_HASH_MOD))
    return jnp.where(m < 0, m + _HASH_MOD, m)


def _partner(a, j, bit0):
    d = 1 << j
    if j < 7:
        plus, minus = jnp.roll(a, -d, axis=1), jnp.roll(a, d, axis=1)
    else:
        dd = d >> 7
        plus, minus = jnp.roll(a, -dd, axis=0), jnp.roll(a, dd, axis=0)
    return jnp.where(bit0, plus, minus)


def _body(xt_ref, w_ref, o_ref, h_scr):
    n = xt_ref.shape[1]
    rows = n // 128
    log_n = n.bit_length() - 1
    w = w_ref[...]

    for r in range(rows):
        xtb = xt_ref[:, pl.ds(r * 128, 128)].astype(jnp.int32)
        s = jnp.sum((xtb + 1) * w, axis=0, keepdims=True)
        h_scr[r:r + 1, :] = _pymod(s)

    row = jax.lax.broadcasted_iota(jnp.int32, (rows, 128), 0)
    lane = jax.lax.broadcasted_iota(jnp.int32, (rows, 128), 1)
    flat = row * 128 + lane

    fs = [flat >> j for j in range(log_n + 1)]
    bit0s = [(fs[j] & 1) == 0 for j in range(log_n + 1)]
    v = h_scr[...]
    u = flat
    for k in range(1, log_n + 1):
        for j in range(k - 1, -1, -1):
            bit0 = bit0s[j]
            take_min = ((fs[k] ^ fs[j]) & 1) == 0
            pv = _partner(v, j, bit0)
            pu = _partner(u, j, bit0)
            vnew = jnp.where(take_min, jnp.minimum(v, pv), jnp.maximum(v, pv))
            u = jnp.where(vnew != v, pu, u)
            v = vnew

    vr = jnp.roll(v, 1, axis=1)
    vprev = jnp.where(lane == 0, jnp.roll(vr, 1, axis=0), vr)
    flags = jnp.where((v != vprev) & (flat != 0), 1, 0).astype(jnp.int32)
    p = flags
    for s_ in (1, 2, 4, 8, 16, 32, 64):
        p = p + jnp.where(lane >= s_, jnp.roll(p, s_, axis=1), 0)
    rowsum = p[:, 127:128]
    ri = jax.lax.broadcasted_iota(jnp.int32, (rows, 1), 0)
    c = rowsum
    s_ = 1
    while s_ < rows:
        c = c + jnp.where(ri >= s_, jnp.roll(c, s_, axis=0), 0)
        s_ *= 2
    cexcl = jnp.where(ri >= 1, jnp.roll(c, 1, axis=0), 0)
    rank = p + cexcl

    q = u * n + rank
    for k in range(1, log_n + 1):
        for j in range(k - 1, -1, -1):
            bit0 = bit0s[j]
            take_min = ((fs[k] ^ fs[j]) & 1) == 0
            pq = _partner(q, j, bit0)
            q = jnp.where(take_min, jnp.minimum(q, pq), jnp.maximum(q, pq))

    o_ref[...] = q & (n - 1)


def kernel(stacked_raw_attributes, blocks_mask):
    x = stacked_raw_attributes
    n, n_attrs = x.shape
    rows = n // 128

    rng = np.random.default_rng(1234)
    weights = jnp.asarray(
        rng.integers(1, _HASH_MOD, size=(n_attrs,), dtype=np.int64).astype(np.int32) | 1
    )
    k_idx = jax.random.split(jax.random.key(42), 4)[0]
    random_index = jax.random.randint(k_idx, (), 0, blocks_mask.shape[0])
    chosen = blocks_mask[random_index]
    mw = jnp.where(chosen, weights, 0).astype(jnp.int32).reshape(n_attrs, 1)

    out = pl.pallas_call(
        _body,
        out_shape=jax.ShapeDtypeStruct((rows, 128), jnp.int32),
        scratch_shapes=[pltpu.VMEM((rows, 128), jnp.int32)],
    )(x.astype(jnp.int8).T, mw)

    return out.reshape(n)


# speedup vs baseline: 1.0886x; 1.0886x over previous
import numpy as np
import jax
import jax.numpy as jnp
from jax.experimental import pallas as pl
from jax.experimental.pallas import tpu as pltpu

_HASH_MOD = 2**31 - 1


def _pymod(s):
    m = jax.lax.rem(s, jnp.int32(_HASH_MOD))
    return jnp.where(m < 0, m + _HASH_MOD, m)


def _partner(a, j, bit0):
    d = 1 << j
    if j < 7:
        plus, minus = jnp.roll(a, -d, axis=1), jnp.roll(a, d, axis=1)
    else:
        dd = d >> 7
        plus, minus = jnp.roll(a, -dd, axis=0), jnp.roll(a, dd, axis=0)
    return jnp.where(bit0, plus, minus)


def _body(xt_ref, w_ref, o_ref, h_scr):
    n = xt_ref.shape[1]
    rows = n // 128
    log_n = n.bit_length() - 1
    w = w_ref[...]

    for r in range(rows):
        xtb = xt_ref[:, pl.ds(r * 128, 128)]
        s = jnp.sum((xtb + 1) * w, axis=0, keepdims=True)
        h_scr[r:r + 1, :] = _pymod(s)

    row = jax.lax.broadcasted_iota(jnp.int32, (rows, 128), 0)
    lane = jax.lax.broadcasted_iota(jnp.int32, (rows, 128), 1)
    flat = row * 128 + lane

    fs = [flat >> j for j in range(log_n + 1)]
    bit0s = [(fs[j] & 1) == 0 for j in range(log_n + 1)]
    v = h_scr[...]
    u = flat
    for k in range(1, log_n + 1):
        for j in range(k - 1, -1, -1):
            bit0 = bit0s[j]
            take_min = ((fs[k] ^ fs[j]) & 1) == 0
            pv = _partner(v, j, bit0)
            pu = _partner(u, j, bit0)
            vnew = jnp.where(take_min, jnp.minimum(v, pv), jnp.maximum(v, pv))
            u = jnp.where(vnew != v, pu, u)
            v = vnew

    vr = jnp.roll(v, 1, axis=1)
    vprev = jnp.where(lane == 0, jnp.roll(vr, 1, axis=0), vr)
    flags = jnp.where((v != vprev) & (flat != 0), 1, 0).astype(jnp.int32)
    p = flags
    for s_ in (1, 2, 4, 8, 16, 32, 64):
        p = p + jnp.where(lane >= s_, jnp.roll(p, s_, axis=1), 0)
    rowsum = p[:, 127:128]
    ri = jax.lax.broadcasted_iota(jnp.int32, (rows, 1), 0)
    c = rowsum
    s_ = 1
    while s_ < rows:
        c = c + jnp.where(ri >= s_, jnp.roll(c, s_, axis=0), 0)
        s_ *= 2
    cexcl = jnp.where(ri >= 1, jnp.roll(c, 1, axis=0), 0)
    rank = p + cexcl

    q = u * n + rank
    for k in range(1, log_n + 1):
        for j in range(k - 1, -1, -1):
            bit0 = bit0s[j]
            take_min = ((fs[k] ^ fs[j]) & 1) == 0
            pq = _partner(q, j, bit0)
            q = jnp.where(take_min, jnp.minimum(q, pq), jnp.maximum(q, pq))

    o_ref[...] = q & (n - 1)


def kernel(stacked_raw_attributes, blocks_mask):
    x = stacked_raw_attributes
    n, n_attrs = x.shape
    rows = n // 128

    rng = np.random.default_rng(1234)
    weights = jnp.asarray(
        rng.integers(1, _HASH_MOD, size=(n_attrs,), dtype=np.int64).astype(np.int32) | 1
    )
    k_idx = jax.random.split(jax.random.key(42), 4)[0]
    random_index = jax.random.randint(k_idx, (), 0, blocks_mask.shape[0])
    chosen = blocks_mask[random_index]
    mw = jnp.where(chosen, weights, 0).astype(jnp.int32).reshape(n_attrs, 1)

    out = pl.pallas_call(
        _body,
        out_shape=jax.ShapeDtypeStruct((rows, 128), jnp.int32),
        scratch_shapes=[pltpu.VMEM((rows, 128), jnp.int32)],
    )(x.T, mw)

    return out.reshape(n)
